# baseline (speedup 1.0000x reference)
import functools

import jax
import jax.numpy as jnp
from jax import lax
from jax.experimental import pallas as pl
from jax.experimental.pallas import tpu as pltpu
from jax.experimental.pallas import tpu_sc as plsc

F32 = jnp.float32
BF16 = jnp.bfloat16
I32 = jnp.int32

_SIGMA = ([v for pair in zip(range(0, 16), range(16, 32)) for v in pair]
          + [v for pair in zip(range(32, 48), range(48, 64)) for v in pair])
_MW2_PERM = (list(range(0, 32, 2)) + list(range(1, 32, 2))
             + list(range(32, 64, 2)) + list(range(33, 64, 2)))

NP = 10240
EP = 327680
NW = 32
EPT = EP // NW
K2 = 256
CH2 = EPT // K2
NPT = NP // 16


def _prep_body(x_ref, w_ref, asw_ref, adw_ref, p_ref, h_ref, hb_ref,
               as_ref, ad_ref, c_ref):
    h = jnp.dot(x_ref[...], w_ref[...], preferred_element_type=F32)
    h_ref[...] = h
    hb_ref[...] = jnp.dot(h, p_ref[...],
                          preferred_element_type=F32).astype(BF16)
    asv = jnp.sum(h * asw_ref[...], axis=1, keepdims=True)
    adv = jnp.sum(h * adw_ref[...], axis=1, keepdims=True)
    as_ref[...] = asv
    ad_ref[...] = adv
    cval = jnp.max(asv) + jnp.max(adv)
    cval = jnp.maximum(cval, 0.2 * cval)
    c_ref[...] = jnp.full((1, 16), cval, F32)


def _prep(x, w, a_src, a_dst):
    return pl.pallas_call(
        _prep_body,
        out_shape=[
            jax.ShapeDtypeStruct((NP, 64), F32),
            jax.ShapeDtypeStruct((NP, 64), BF16),
            jax.ShapeDtypeStruct((NP, 1), F32),
            jax.ShapeDtypeStruct((NP, 1), F32),
            jax.ShapeDtypeStruct((1, 16), F32),
        ],
    )(x, w, a_src.reshape(1, 64), a_dst.reshape(1, 64),
      jnp.eye(64, dtype=F32)[:, jnp.array(_SIGMA)])


def _combine_body(nnodes, matmuls, acc_ref, den_ref, h_ref, as_ref, ad_ref,
                  c_ref, b_ref, *rest):
    w_refs = rest[:matmuls]
    out_refs = rest[matmuls:matmuls + matmuls]
    cval = c_ref[0, 0]
    s = as_ref[...] + ad_ref[...]
    wself = jnp.exp(jnp.maximum(s, 0.2 * s) - cval)
    dent = lax.dot_general(den_ref[...], jnp.ones((NW, 1), F32),
                           (((0,), (0,)), ((), ())),
                           preferred_element_type=F32)
    dent = dent + wself + 1e-16
    a = acc_ref[...]
    acct = a[0] + a[1] + wself * h_ref[...]
    g = jnp.maximum(acct / dent + b_ref[...], 0.0)
    rows = lax.broadcasted_iota(I32, (NP, 1), 0)
    g = jnp.where(rows < nnodes, g, 0.0)
    for w_ref, out_ref in zip(w_refs, out_refs):
        out_ref[...] = jnp.dot(g, w_ref[...], preferred_element_type=F32)


def _combine_next(nnodes, acc, den, h, asv, adv, c, b, w2, a_src2, a_dst2):
    def body(acc_ref, den_ref, h_ref, as_ref, ad_ref, c_ref, b_ref, w_ref,
             asw_ref, adw_ref, p_ref, hn_ref, hnb_ref, as2_ref, ad2_ref,
             c2_ref):
        cval = c_ref[0, 0]
        s = as_ref[...] + ad_ref[...]
        wself = jnp.exp(jnp.maximum(s, 0.2 * s) - cval)
        dent = lax.dot_general(den_ref[...], jnp.ones((NW, 1), F32),
                               (((0,), (0,)), ((), ())),
                               preferred_element_type=F32)
        dent = dent + wself + 1e-16
        a = acc_ref[...]
        acct = a[0] + a[1] + wself * h_ref[...]
        g = jnp.maximum(acct / dent + b_ref[...], 0.0)
        rows = lax.broadcasted_iota(I32, (NP, 1), 0)
        g = jnp.where(rows < nnodes, g, 0.0)
        hn = jnp.dot(g, w_ref[...], preferred_element_type=F32)
        hn_ref[...] = hn
        hnb_ref[...] = jnp.dot(hn, p_ref[...],
                               preferred_element_type=F32).astype(BF16)
        asv2 = jnp.sum(hn * asw_ref[...], axis=1, keepdims=True)
        adv2 = jnp.sum(hn * adw_ref[...], axis=1, keepdims=True)
        as2_ref[...] = asv2
        ad2_ref[...] = adv2
        cv2 = jnp.max(asv2) + jnp.max(adv2)
        cv2 = jnp.maximum(cv2, 0.2 * cv2)
        c2_ref[...] = jnp.full((1, 16), cv2, F32)

    return pl.pallas_call(
        body,
        out_shape=[
            jax.ShapeDtypeStruct((NP, 64), F32),
            jax.ShapeDtypeStruct((NP, 64), BF16),
            jax.ShapeDtypeStruct((NP, 1), F32),
            jax.ShapeDtypeStruct((NP, 1), F32),
            jax.ShapeDtypeStruct((1, 16), F32),
        ],
    )(acc, den, h, asv, adv, c, b.reshape(1, 64), w2,
      a_src2.reshape(1, 64), a_dst2.reshape(1, 64),
      jnp.eye(64, dtype=F32)[:, jnp.array(_SIGMA)])


def _combine_proj(nnodes, acc, den, h, asv, adv, c, b, w_u, w_v):
    def body(acc_ref, den_ref, h_ref, as_ref, ad_ref, c_ref, b_ref,
             wu_ref, wv_ref, g_ref, pu_ref, pv_ref):
        cval = c_ref[0, 0]
        s = as_ref[...] + ad_ref[...]
        wself = jnp.exp(jnp.maximum(s, 0.2 * s) - cval)
        dent = lax.dot_general(den_ref[...], jnp.ones((NW, 1), F32),
                               (((0,), (0,)), ((), ())),
                               preferred_element_type=F32)
        dent = dent + wself + 1e-16
        a = acc_ref[...]
        acct = a[0] + a[1] + wself * h_ref[...]
        g = jnp.maximum(acct / dent + b_ref[...], 0.0)
        rows = lax.broadcasted_iota(I32, (NP, 1), 0)
        g = jnp.where(rows < nnodes, g, 0.0)
        g_ref[...] = g
        pu_ref[...] = jnp.dot(
            g, wu_ref[...], preferred_element_type=F32).astype(BF16)
        pv_ref[...] = jnp.dot(
            g, wv_ref[...], preferred_element_type=F32).astype(BF16)

    return pl.pallas_call(
        body,
        out_shape=[
            jax.ShapeDtypeStruct((NP, 64), F32),
            jax.ShapeDtypeStruct((NP, 64), BF16),
            jax.ShapeDtypeStruct((NP, 64), BF16),
        ],
    )(acc, den, h, asv, adv, c, b.reshape(1, 64), w_u, w_v)


def _pe_body(ea_ref, w_ref, b_ref, o_ref):
    o_ref[...] = (jnp.dot(ea_ref[...], w_ref[...],
                          preferred_element_type=F32)
                  + b_ref[...]).astype(BF16)


def _edge_proj(ea_pad, w_e, mb1):
    bm = 10240
    return pl.pallas_call(
        _pe_body,
        grid=(EP // bm,),
        in_specs=[pl.BlockSpec((bm, 16), lambda i: (i, 0)),
                  pl.BlockSpec((16, 64), lambda i: (0, 0)),
                  pl.BlockSpec((1, 64), lambda i: (0, 0))],
        out_specs=pl.BlockSpec((bm, 64), lambda i: (i, 0)),
        out_shape=jax.ShapeDtypeStruct((EP, 64), BF16),
    )(ea_pad, w_e, mb1.reshape(1, 64))


_MESH = plsc.VectorSubcoreMesh(core_axis_name="c", subcore_axis_name="s")
_SC_PARAMS = pltpu.CompilerParams(needs_layout_passes=False,
                                  use_tc_tiling_on_sc=False)


@functools.partial(
    pl.kernel,
    out_type=[
        jax.ShapeDtypeStruct((2, NP, 64), F32),
        jax.ShapeDtypeStruct((NW, NP), F32),
    ],
    mesh=_MESH,
    scratch_types=[
        pltpu.VMEM((2, 128), I32),
        pltpu.VMEM((2, 128), I32),
        pltpu.VMEM((2, 128), I32),
        pltpu.VMEM((2, 128), I32),
        pltpu.VMEM((K2, 64), BF16),
        pltpu.VMEM((K2, 64), BF16),
        pltpu.VMEM((K2, 64), F32),
        pltpu.VMEM((K2, 64), F32),
        pltpu.VMEM((NP,), F32),
        pltpu.VMEM((NP,), F32),
        pltpu.VMEM((NP,), F32),
        pltpu.VMEM((K2,), F32),
        pltpu.VMEM((16,), F32),
        pltpu.VMEM_SHARED((NP, 64), F32),
        pltpu.SemaphoreType.DMA,
        pltpu.SemaphoreType.DMA,
    ],
    compiler_params=_SC_PARAMS,
)
def _gat_edges(src2_hbm, dst2_hbm, h_hbm, as_hbm,
               ad_hbm, c_hbm, zrow_hbm, zflat_hbm, acc_out, den_out,
               src2a, dst2a, src2b, dst2b, rba, rbb, rowsa, rowsb, as_v,
               ad_v, den_v, w_v, c_v, acc_sh, sem, sem2):
    cid = lax.axis_index("c")
    sid = lax.axis_index("s")
    wid = sid * 2 + cid
    srcbufs = (src2a, src2b)
    dstbufs = (dst2a, dst2b)
    bfbufs = (rba, rbb)
    rowbufs = (rowsa, rowsb)

    pltpu.sync_copy(as_hbm, as_v)
    pltpu.sync_copy(ad_hbm, ad_v)
    pltpu.sync_copy(c_hbm, c_v)
    pltpu.sync_copy(zflat_hbm, den_v)
    pltpu.sync_copy(zrow_hbm, acc_sh.at[pl.ds(sid * NPT, NPT)])
    plsc.subcore_barrier()
    cvec = c_v[...]

    def fire(cg, p):
        pltpu.sync_copy(src2_hbm.at[cg], srcbufs[p])
        pltpu.sync_copy(dst2_hbm.at[cg], dstbufs[p])
        for j in range(2):
            pltpu.async_copy(h_hbm.at[srcbufs[p].at[j]],
                             bfbufs[p].at[pl.ds(j * 128, 128)], sem)

    fire(wid * CH2, 0)

    def drain_scatter(p):
        for j in range(2):
            pltpu.make_async_copy(rowbufs[p].at[pl.ds(j * 128, 128)],
                                  acc_sh.at[dstbufs[p].at[j]], sem2).wait()

    def pair(i, carry):
        for b in range(2):
            ci = 2 * i + b
            cg = wid * CH2 + ci
            for j in range(2):
                pltpu.make_async_copy(
                    h_hbm.at[srcbufs[b].at[j]],
                    bfbufs[b].at[pl.ds(j * 128, 128)], sem).wait()

            @pl.when(ci > 0)
            def _():
                drain_scatter(1 - b)

            @pl.when(ci + 1 < CH2)
            def _():
                fire(cg + 1, 1 - b)

            def grp(g, inner):
                jj = g // 8
                oo = (g % 8) * 16
                s16 = srcbufs[b][jj, pl.ds(oo, 16)]
                d16 = dstbufs[b][jj, pl.ds(oo, 16)]
                a1 = plsc.load_gather(as_v, [s16])
                a2 = plsc.load_gather(ad_v, [d16])
                t = a1 + a2
                e = jnp.maximum(t, 0.2 * t)
                w = jnp.exp(e - cvec)
                plsc.addupdate_scatter(den_v, [d16], w)
                w_v[pl.ds(g * 16, 16)] = w
                return inner

            lax.fori_loop(0, K2 // 16, grp, 0)

            @plsc.parallel_loop(0, K2, unroll=4)
            def _(r):
                wb = plsc.load_gather(w_v, [jnp.full((16,), r, I32)])
                for half in range(2):
                    xh = bfbufs[b][r, pl.ds(half * 32, 32)]
                    ev, od = plsc.unpack(xh, format=plsc.PackFormat.INTERLEAVED)
                    rowbufs[b][r, pl.ds(half * 32, 16)] = ev * wb
                    rowbufs[b][r, pl.ds(half * 32 + 16, 16)] = od * wb
            for j in range(2):
                pltpu.async_copy(rowbufs[b].at[pl.ds(j * 128, 128)],
                                 acc_sh.at[dstbufs[b].at[j]], sem2,
                                 add=True)
        return carry

    lax.fori_loop(0, CH2 // 2, pair, 0)
    drain_scatter(1)
    plsc.subcore_barrier()
    pltpu.sync_copy(acc_sh.at[pl.ds(sid * NPT, NPT)],
                    acc_out.at[cid, pl.ds(sid * NPT, NPT)])
    pltpu.sync_copy(den_v, den_out.at[wid])


@functools.partial(
    pl.kernel,
    out_type=jax.ShapeDtypeStruct((EP,), F32),
    mesh=_MESH,
    scratch_types=[
        pltpu.VMEM((2, 128), I32),
        pltpu.VMEM((2, 128), I32),
        pltpu.VMEM((2, 128), I32),
        pltpu.VMEM((2, 128), I32),
        pltpu.VMEM((K2, 64), BF16),
        pltpu.VMEM((K2, 64), BF16),
        pltpu.VMEM((K2, 64), BF16),
        pltpu.VMEM((K2, 64), BF16),
        pltpu.VMEM((K2, 64), BF16),
        pltpu.VMEM((K2, 64), BF16),
        pltpu.VMEM((64,), F32),
        pltpu.VMEM((16,), F32),
        pltpu.VMEM((K2,), F32),
        pltpu.SemaphoreType.DMA,
    ],
    compiler_params=_SC_PARAMS,
)
def _edge_mlp(src2_hbm, dst2_hbm, pu_hbm, pv_hbm, pe_hbm, w2_hbm, b2_hbm,
              out_hbm, srcma, dstma, srcmb, dstmb, pua, pva, pea, pub, pvb,
              peb, w2_v, b2_v, lo_v, sem):
    cid = lax.axis_index("c")
    sid = lax.axis_index("s")
    wid = sid * 2 + cid
    srcbufs = (srcma, srcmb)
    dstbufs = (dstma, dstmb)
    pubufs = (pua, pub)
    pvbufs = (pva, pvb)
    pebufs = (pea, peb)

    pltpu.sync_copy(w2_hbm, w2_v)
    pltpu.sync_copy(b2_hbm, b2_v)
    mvec = [w2_v[pl.ds(q * 16, 16)] for q in range(4)]
    bvec = b2_v[...]

    def fire(ci, p):
        cg = wid * CH2 + ci
        base = wid * EPT + ci * K2
        pltpu.sync_copy(src2_hbm.at[cg], srcbufs[p])
        pltpu.sync_copy(dst2_hbm.at[cg], dstbufs[p])
        for j in range(2):
            pltpu.async_copy(pu_hbm.at[srcbufs[p].at[j]],
                             pubufs[p].at[pl.ds(j * 128, 128)], sem)
            pltpu.async_copy(pv_hbm.at[dstbufs[p].at[j]],
                             pvbufs[p].at[pl.ds(j * 128, 128)], sem)
        pltpu.async_copy(pe_hbm.at[pl.ds(base, K2)], pebufs[p], sem)

    fire(0, 0)

    def pair(i, carry):
        for b in range(2):
            ci = 2 * i + b
            base = wid * EPT + ci * K2
            for j in range(2):
                pltpu.make_async_copy(
                    pu_hbm.at[srcbufs[b].at[j]],
                    pubufs[b].at[pl.ds(j * 128, 128)], sem).wait()
                pltpu.make_async_copy(
                    pv_hbm.at[dstbufs[b].at[j]],
                    pvbufs[b].at[pl.ds(j * 128, 128)], sem).wait()
            pltpu.make_async_copy(pe_hbm.at[pl.ds(base, K2)],
                                  pebufs[b], sem).wait()

            @pl.when(ci + 1 < CH2)
            def _():
                fire(ci + 1, 1 - b)

            def init(g, inner):
                lo_v[pl.ds(g * 16, 16)] = bvec
                return inner

            lax.fori_loop(0, K2 // 16, init, 0)

            @plsc.parallel_loop(0, K2, unroll=4)
            def _(r):
                acc = None
                for half in range(2):
                    ds = pl.ds(half * 32, 32)
                    ua, ub = plsc.unpack(pubufs[b][r, ds],
                                         format=plsc.PackFormat.INTERLEAVED)
                    va, vb = plsc.unpack(pvbufs[b][r, ds],
                                         format=plsc.PackFormat.INTERLEAVED)
                    ea, eb = plsc.unpack(pebufs[b][r, ds],
                                         format=plsc.PackFormat.INTERLEAVED)
                    for pp, vv, ee, q in ((ua, va, ea, 2 * half),
                                          (ub, vb, eb, 2 * half + 1)):
                        hid = jnp.maximum(pp + vv + ee, 0.0)
                        term = hid * mvec[q]
                        acc = term if acc is None else acc + term
                plsc.addupdate_scatter(lo_v, [jnp.full((16,), r, I32)], acc)
            pltpu.sync_copy(lo_v, out_hbm.at[pl.ds(base, K2)])
        return carry

    lax.fori_loop(0, CH2 // 2, pair, 0)


def kernel(x, edge_index, edge_attr, W1, a_src1, a_dst1, b1, W2, a_src2,
           a_dst2, b2, mW1, mb1, mW2, mb2):
    n = x.shape[0]
    e = edge_index.shape[1]

    src = edge_index[0]
    dst = edge_index[1]
    pad = jnp.full((EP - e,), n, I32)
    srcf = jnp.concatenate([src, pad])
    dstf = jnp.concatenate([dst, pad])
    src2 = srcf.reshape(EP // K2, 2, 128)
    dst2 = dstf.reshape(EP // K2, 2, 128)

    x_pad = jnp.pad(x, ((0, NP - n), (0, 0)))
    ea_pad = jnp.pad(edge_attr, ((0, EP - e), (0, 0)))
    zrow = jnp.zeros((NPT, 64), F32)
    zflat = jnp.zeros((NP,), F32)

    h1, h1b, as1, ad1, c1 = _prep(x_pad, W1, a_src1, a_dst1)
    c1v = c1.reshape(16)
    acc1, den1 = _gat_edges(src2, dst2, h1b, as1.reshape(NP),
                            ad1.reshape(NP), c1v, zrow, zflat)
    h2, h2b, as2, ad2, c2 = _combine_next(n, acc1, den1, h1, as1, ad1, c1,
                                          b1, W2, a_src2, a_dst2)
    c2v = c2.reshape(16)
    acc2, den2 = _gat_edges(src2, dst2, h2b, as2.reshape(NP),
                            ad2.reshape(NP), c2v, zrow, zflat)
    g2, pu, pv = _combine_proj(n, acc2, den2, h2, as2, ad2, c2, b2,
                               mW1[:64], mW1[64:128])
    pe = _edge_proj(ea_pad, mW1[128:], mb1)
    mw2r = mW2.reshape(64)[jnp.array(_MW2_PERM)]
    logits_pad = _edge_mlp(src2, dst2, pu, pv, pe, mw2r,
                           jnp.full((16,), mb2[0], F32))
    return (logits_pad[:e], g2[:n])


# speedup vs baseline: 1.0272x; 1.0272x over previous
import functools

import jax
import jax.numpy as jnp
from jax import lax
from jax.experimental import pallas as pl
from jax.experimental.pallas import tpu as pltpu
from jax.experimental.pallas import tpu_sc as plsc

F32 = jnp.float32
BF16 = jnp.bfloat16
I32 = jnp.int32

_SIGMA = ([v for pair in zip(range(0, 16), range(16, 32)) for v in pair]
          + [v for pair in zip(range(32, 48), range(48, 64)) for v in pair])
_MW2_PERM = (list(range(0, 32, 2)) + list(range(1, 32, 2))
             + list(range(32, 64, 2)) + list(range(33, 64, 2)))

NP = 10240
EP = 327680
NW = 32
EPT = EP // NW
K2 = 256
CH2 = EPT // K2
NPT = NP // 16


def _prep_body(x_ref, w_ref, asw_ref, adw_ref, p_ref, h_ref, hb_ref,
               as_ref, ad_ref, c_ref):
    h = jnp.dot(x_ref[...], w_ref[...], preferred_element_type=F32)
    h_ref[...] = h
    hb_ref[...] = jnp.dot(h, p_ref[...],
                          preferred_element_type=F32).astype(BF16)
    asv = jnp.sum(h * asw_ref[...], axis=1, keepdims=True)
    adv = jnp.sum(h * adw_ref[...], axis=1, keepdims=True)
    as_ref[...] = asv
    ad_ref[...] = adv
    cval = jnp.max(asv) + jnp.max(adv)
    cval = jnp.maximum(cval, 0.2 * cval)
    c_ref[...] = jnp.full((1, 16), cval, F32)


def _prep(x, w, a_src, a_dst):
    return pl.pallas_call(
        _prep_body,
        out_shape=[
            jax.ShapeDtypeStruct((NP, 64), F32),
            jax.ShapeDtypeStruct((NP, 64), BF16),
            jax.ShapeDtypeStruct((NP, 1), F32),
            jax.ShapeDtypeStruct((NP, 1), F32),
            jax.ShapeDtypeStruct((1, 16), F32),
        ],
    )(x, w, a_src.reshape(1, 64), a_dst.reshape(1, 64),
      jnp.eye(64, dtype=F32)[:, jnp.array(_SIGMA)])


def _combine_body(nnodes, matmuls, acc_ref, den_ref, h_ref, as_ref, ad_ref,
                  c_ref, b_ref, *rest):
    w_refs = rest[:matmuls]
    out_refs = rest[matmuls:matmuls + matmuls]
    cval = c_ref[0, 0]
    s = as_ref[...] + ad_ref[...]
    wself = jnp.exp(jnp.maximum(s, 0.2 * s) - cval)
    dent = lax.dot_general(den_ref[...], jnp.ones((NW, 1), F32),
                           (((0,), (0,)), ((), ())),
                           preferred_element_type=F32)
    dent = dent + wself + 1e-16
    a = acc_ref[...]
    acct = a[0] + a[1] + wself * h_ref[...]
    g = jnp.maximum(acct / dent + b_ref[...], 0.0)
    rows = lax.broadcasted_iota(I32, (NP, 1), 0)
    g = jnp.where(rows < nnodes, g, 0.0)
    for w_ref, out_ref in zip(w_refs, out_refs):
        out_ref[...] = jnp.dot(g, w_ref[...], preferred_element_type=F32)


def _combine_next(nnodes, acc, den, h, asv, adv, c, b, w2, a_src2, a_dst2):
    def body(acc_ref, den_ref, h_ref, as_ref, ad_ref, c_ref, b_ref, w_ref,
             asw_ref, adw_ref, p_ref, hn_ref, hnb_ref, as2_ref, ad2_ref,
             c2_ref):
        cval = c_ref[0, 0]
        s = as_ref[...] + ad_ref[...]
        wself = jnp.exp(jnp.maximum(s, 0.2 * s) - cval)
        dent = lax.dot_general(den_ref[...], jnp.ones((NW, 1), F32),
                               (((0,), (0,)), ((), ())),
                               preferred_element_type=F32)
        dent = dent + wself + 1e-16
        a = acc_ref[...]
        acct = a[0] + a[1] + wself * h_ref[...]
        g = jnp.maximum(acct / dent + b_ref[...], 0.0)
        rows = lax.broadcasted_iota(I32, (NP, 1), 0)
        g = jnp.where(rows < nnodes, g, 0.0)
        hn = jnp.dot(g, w_ref[...], preferred_element_type=F32)
        hn_ref[...] = hn
        hnb_ref[...] = jnp.dot(hn, p_ref[...],
                               preferred_element_type=F32).astype(BF16)
        asv2 = jnp.sum(hn * asw_ref[...], axis=1, keepdims=True)
        adv2 = jnp.sum(hn * adw_ref[...], axis=1, keepdims=True)
        as2_ref[...] = asv2
        ad2_ref[...] = adv2
        cv2 = jnp.max(asv2) + jnp.max(adv2)
        cv2 = jnp.maximum(cv2, 0.2 * cv2)
        c2_ref[...] = jnp.full((1, 16), cv2, F32)

    return pl.pallas_call(
        body,
        out_shape=[
            jax.ShapeDtypeStruct((NP, 64), F32),
            jax.ShapeDtypeStruct((NP, 64), BF16),
            jax.ShapeDtypeStruct((NP, 1), F32),
            jax.ShapeDtypeStruct((NP, 1), F32),
            jax.ShapeDtypeStruct((1, 16), F32),
        ],
    )(acc, den, h, asv, adv, c, b.reshape(1, 64), w2,
      a_src2.reshape(1, 64), a_dst2.reshape(1, 64),
      jnp.eye(64, dtype=F32)[:, jnp.array(_SIGMA)])


def _combine_proj(nnodes, acc, den, h, asv, adv, c, b, w_u, w_v):
    def body(acc_ref, den_ref, h_ref, as_ref, ad_ref, c_ref, b_ref,
             wu_ref, wv_ref, g_ref, pu_ref, pv_ref):
        cval = c_ref[0, 0]
        s = as_ref[...] + ad_ref[...]
        wself = jnp.exp(jnp.maximum(s, 0.2 * s) - cval)
        dent = lax.dot_general(den_ref[...], jnp.ones((NW, 1), F32),
                               (((0,), (0,)), ((), ())),
                               preferred_element_type=F32)
        dent = dent + wself + 1e-16
        a = acc_ref[...]
        acct = a[0] + a[1] + wself * h_ref[...]
        g = jnp.maximum(acct / dent + b_ref[...], 0.0)
        rows = lax.broadcasted_iota(I32, (NP, 1), 0)
        g = jnp.where(rows < nnodes, g, 0.0)
        g_ref[...] = g
        pu_ref[...] = jnp.dot(
            g, wu_ref[...], preferred_element_type=F32).astype(BF16)
        pv_ref[...] = jnp.dot(
            g, wv_ref[...], preferred_element_type=F32).astype(BF16)

    return pl.pallas_call(
        body,
        out_shape=[
            jax.ShapeDtypeStruct((NP, 64), F32),
            jax.ShapeDtypeStruct((NP, 64), BF16),
            jax.ShapeDtypeStruct((NP, 64), BF16),
        ],
    )(acc, den, h, asv, adv, c, b.reshape(1, 64), w_u, w_v)


def _pe_body(ea_ref, w_ref, b_ref, o_ref):
    o_ref[...] = (jnp.dot(ea_ref[...], w_ref[...],
                          preferred_element_type=F32)
                  + b_ref[...]).astype(BF16)


def _edge_proj(ea_pad, w_e, mb1):
    bm = 10240
    return pl.pallas_call(
        _pe_body,
        grid=(EP // bm,),
        in_specs=[pl.BlockSpec((bm, 16), lambda i: (i, 0)),
                  pl.BlockSpec((16, 64), lambda i: (0, 0)),
                  pl.BlockSpec((1, 64), lambda i: (0, 0))],
        out_specs=pl.BlockSpec((bm, 64), lambda i: (i, 0)),
        out_shape=jax.ShapeDtypeStruct((EP, 64), BF16),
    )(ea_pad, w_e, mb1.reshape(1, 64))


_MESH = plsc.VectorSubcoreMesh(core_axis_name="c", subcore_axis_name="s")
_SC_PARAMS = pltpu.CompilerParams(needs_layout_passes=False,
                                  use_tc_tiling_on_sc=False)


@functools.partial(
    pl.kernel,
    out_type=[
        jax.ShapeDtypeStruct((2, NP, 64), F32),
        jax.ShapeDtypeStruct((NW, NP), F32),
    ],
    mesh=_MESH,
    scratch_types=[
        pltpu.VMEM((2, 128), I32),
        pltpu.VMEM((2, 128), I32),
        pltpu.VMEM((2, 128), I32),
        pltpu.VMEM((2, 128), I32),
        pltpu.VMEM((K2, 64), BF16),
        pltpu.VMEM((K2, 64), BF16),
        pltpu.VMEM((K2, 64), F32),
        pltpu.VMEM((K2, 64), F32),
        pltpu.VMEM((NP,), F32),
        pltpu.VMEM((NP,), F32),
        pltpu.VMEM((NP,), F32),
        pltpu.VMEM((K2,), F32),
        pltpu.VMEM((16,), F32),
        pltpu.VMEM_SHARED((NP, 64), F32),
        pltpu.SemaphoreType.DMA,
        pltpu.SemaphoreType.DMA,
    ],
    compiler_params=_SC_PARAMS,
)
def _gat_edges(src2_hbm, dst2_hbm, h_hbm, as_hbm,
               ad_hbm, c_hbm, zrow_hbm, zflat_hbm, acc_out, den_out,
               src2a, dst2a, src2b, dst2b, rba, rbb, rowsa, rowsb, as_v,
               ad_v, den_v, w_v, c_v, acc_sh, sem, sem2):
    cid = lax.axis_index("c")
    sid = lax.axis_index("s")
    wid = sid * 2 + cid
    srcbufs = (src2a, src2b)
    dstbufs = (dst2a, dst2b)
    bfbufs = (rba, rbb)
    rowbufs = (rowsa, rowsb)

    pltpu.sync_copy(as_hbm, as_v)
    pltpu.sync_copy(ad_hbm, ad_v)
    pltpu.sync_copy(c_hbm, c_v)
    pltpu.sync_copy(zflat_hbm, den_v)
    pltpu.sync_copy(zrow_hbm, acc_sh.at[pl.ds(sid * NPT, NPT)])
    plsc.subcore_barrier()
    cvec = c_v[...]

    def fire(cg, p):
        pltpu.sync_copy(src2_hbm.at[cg], srcbufs[p])
        pltpu.sync_copy(dst2_hbm.at[cg], dstbufs[p])
        for j in range(2):
            pltpu.async_copy(h_hbm.at[srcbufs[p].at[j]],
                             bfbufs[p].at[pl.ds(j * 128, 128)], sem)

    fire(wid * CH2, 0)

    def drain_scatter(p):
        for j in range(2):
            pltpu.make_async_copy(rowbufs[p].at[pl.ds(j * 128, 128)],
                                  acc_sh.at[dstbufs[p].at[j]], sem2).wait()

    def pair(i, carry):
        for b in range(2):
            ci = 2 * i + b
            cg = wid * CH2 + ci
            for j in range(2):
                pltpu.make_async_copy(
                    h_hbm.at[srcbufs[b].at[j]],
                    bfbufs[b].at[pl.ds(j * 128, 128)], sem).wait()

            @pl.when(ci > 0)
            def _():
                drain_scatter(1 - b)

            @pl.when(ci + 1 < CH2)
            def _():
                fire(cg + 1, 1 - b)

            def grp(g, inner):
                jj = g // 8
                oo = (g % 8) * 16
                s16 = srcbufs[b][jj, pl.ds(oo, 16)]
                d16 = dstbufs[b][jj, pl.ds(oo, 16)]
                a1 = plsc.load_gather(as_v, [s16])
                a2 = plsc.load_gather(ad_v, [d16])
                t = a1 + a2
                e = jnp.maximum(t, 0.2 * t)
                w = jnp.exp(e - cvec)
                plsc.addupdate_scatter(den_v, [d16], w)
                w_v[pl.ds(g * 16, 16)] = w
                return inner

            lax.fori_loop(0, K2 // 16, grp, 0)

            @plsc.parallel_loop(0, K2, unroll=4)
            def _(r):
                wb = plsc.load_gather(w_v, [jnp.full((16,), r, I32)])
                for half in range(2):
                    xh = bfbufs[b][r, pl.ds(half * 32, 32)]
                    ev, od = plsc.unpack(xh, format=plsc.PackFormat.INTERLEAVED)
                    rowbufs[b][r, pl.ds(half * 32, 16)] = ev * wb
                    rowbufs[b][r, pl.ds(half * 32 + 16, 16)] = od * wb
            for j in range(2):
                pltpu.async_copy(rowbufs[b].at[pl.ds(j * 128, 128)],
                                 acc_sh.at[dstbufs[b].at[j]], sem2,
                                 add=True)
        return carry

    lax.fori_loop(0, CH2 // 2, pair, 0)
    drain_scatter(1)
    plsc.subcore_barrier()
    pltpu.sync_copy(acc_sh.at[pl.ds(sid * NPT, NPT)],
                    acc_out.at[cid, pl.ds(sid * NPT, NPT)])
    pltpu.sync_copy(den_v, den_out.at[wid])


@functools.partial(
    pl.kernel,
    out_type=jax.ShapeDtypeStruct((EP,), F32),
    mesh=_MESH,
    scratch_types=[
        pltpu.VMEM((2, 128), I32),
        pltpu.VMEM((2, 128), I32),
        pltpu.VMEM((2, 128), I32),
        pltpu.VMEM((2, 128), I32),
        pltpu.VMEM((K2, 64), BF16),
        pltpu.VMEM((K2, 64), BF16),
        pltpu.VMEM((K2, 64), BF16),
        pltpu.VMEM((K2, 64), BF16),
        pltpu.VMEM((K2, 64), BF16),
        pltpu.VMEM((K2, 64), BF16),
        pltpu.VMEM((64,), F32),
        pltpu.VMEM((16,), F32),
        pltpu.VMEM((K2,), F32),
        pltpu.SemaphoreType.DMA,
    ],
    compiler_params=_SC_PARAMS,
)
def _edge_mlp(src2_hbm, dst2_hbm, pu_hbm, pv_hbm, pe_hbm, w2_hbm, b2_hbm,
              out_hbm, srcma, dstma, srcmb, dstmb, pua, pva, pea, pub, pvb,
              peb, w2_v, b2_v, lo_v, sem):
    cid = lax.axis_index("c")
    sid = lax.axis_index("s")
    wid = sid * 2 + cid
    srcbufs = (srcma, srcmb)
    dstbufs = (dstma, dstmb)
    pubufs = (pua, pub)
    pvbufs = (pva, pvb)
    pebufs = (pea, peb)

    pltpu.sync_copy(w2_hbm, w2_v)
    pltpu.sync_copy(b2_hbm, b2_v)
    mvec = [w2_v[pl.ds(q * 16, 16)] for q in range(4)]
    bvec = b2_v[...]

    def fire(ci, p):
        cg = wid * CH2 + ci
        base = wid * EPT + ci * K2
        pltpu.sync_copy(src2_hbm.at[cg], srcbufs[p])
        pltpu.sync_copy(dst2_hbm.at[cg], dstbufs[p])
        for j in range(2):
            pltpu.async_copy(pu_hbm.at[srcbufs[p].at[j]],
                             pubufs[p].at[pl.ds(j * 128, 128)], sem)
            pltpu.async_copy(pv_hbm.at[dstbufs[p].at[j]],
                             pvbufs[p].at[pl.ds(j * 128, 128)], sem)
        pltpu.async_copy(pe_hbm.at[pl.ds(base, K2)], pebufs[p], sem)

    fire(0, 0)

    def pair(i, carry):
        for b in range(2):
            ci = 2 * i + b
            base = wid * EPT + ci * K2
            for j in range(2):
                pltpu.make_async_copy(
                    pu_hbm.at[srcbufs[b].at[j]],
                    pubufs[b].at[pl.ds(j * 128, 128)], sem).wait()
                pltpu.make_async_copy(
                    pv_hbm.at[dstbufs[b].at[j]],
                    pvbufs[b].at[pl.ds(j * 128, 128)], sem).wait()
            pltpu.make_async_copy(pe_hbm.at[pl.ds(base, K2)],
                                  pebufs[b], sem).wait()

            @pl.when(ci + 1 < CH2)
            def _():
                fire(ci + 1, 1 - b)

            def init(g, inner):
                lo_v[pl.ds(g * 16, 16)] = bvec
                return inner

            lax.fori_loop(0, K2 // 16, init, 0)

            @plsc.parallel_loop(0, K2, unroll=4)
            def _(r):
                acc = None
                for half in range(2):
                    ds = pl.ds(half * 32, 32)
                    ua, ub = plsc.unpack(pubufs[b][r, ds],
                                         format=plsc.PackFormat.INTERLEAVED)
                    va, vb = plsc.unpack(pvbufs[b][r, ds],
                                         format=plsc.PackFormat.INTERLEAVED)
                    ea, eb = plsc.unpack(pebufs[b][r, ds],
                                         format=plsc.PackFormat.INTERLEAVED)
                    for pp, vv, ee, q in ((ua, va, ea, 2 * half),
                                          (ub, vb, eb, 2 * half + 1)):
                        hid = jnp.maximum(pp + vv + ee, 0.0)
                        term = hid * mvec[q]
                        acc = term if acc is None else acc + term
                plsc.addupdate_scatter(lo_v, [jnp.full((16,), r, I32)], acc)
            pltpu.sync_copy(lo_v, out_hbm.at[pl.ds(base, K2)])
        return carry

    lax.fori_loop(0, CH2 // 2, pair, 0)


def kernel(x, edge_index, edge_attr, W1, a_src1, a_dst1, b1, W2, a_src2,
           a_dst2, b2, mW1, mb1, mW2, mb2):
    n = x.shape[0]
    e = edge_index.shape[1]

    src = edge_index[0]
    dst = edge_index[1]
    pad = jnp.full((EP - e,), n, I32)
    srcf = jnp.concatenate([src, pad])
    dstf = jnp.concatenate([dst, pad])
    src2 = srcf.reshape(EP // K2, 2, 128)
    dst2 = dstf.reshape(EP // K2, 2, 128)

    x_pad = jnp.pad(x, ((0, NP - n), (0, 0)))
    ea_pad = jnp.pad(edge_attr, ((0, EP - e), (0, 0)))
    zrow = jnp.zeros((NPT, 64), F32)
    zflat = jnp.zeros((NP,), F32)

    pe = _edge_proj(ea_pad, mW1[128:], mb1)

    h1, h1b, as1, ad1, c1 = _prep(x_pad, W1, a_src1, a_dst1)
    c1v = c1.reshape(16) + (pe[0, 0] * 0).astype(F32)
    acc1, den1 = _gat_edges(src2, dst2, h1b, as1.reshape(NP),
                            ad1.reshape(NP), c1v, zrow, zflat)
    h2, h2b, as2, ad2, c2 = _combine_next(n, acc1, den1, h1, as1, ad1, c1,
                                          b1, W2, a_src2, a_dst2)
    c2v = c2.reshape(16)
    acc2, den2 = _gat_edges(src2, dst2, h2b, as2.reshape(NP),
                            ad2.reshape(NP), c2v, zrow, zflat)
    g2, pu, pv = _combine_proj(n, acc2, den2, h2, as2, ad2, c2, b2,
                               mW1[:64], mW1[64:128])
    mw2r = mW2.reshape(64)[jnp.array(_MW2_PERM)]
    logits_pad = _edge_mlp(src2, dst2, pu, pv, pe, mw2r,
                           jnp.full((16,), mb2[0], F32))
    return (logits_pad[:e], g2[:n])


# speedup vs baseline: 1.0734x; 1.0450x over previous
import functools

import jax
import jax.numpy as jnp
from jax import lax
from jax.experimental import pallas as pl
from jax.experimental.pallas import tpu as pltpu
from jax.experimental.pallas import tpu_sc as plsc

F32 = jnp.float32
BF16 = jnp.bfloat16
I32 = jnp.int32

_SIGMA = ([v for pair in zip(range(0, 16), range(16, 32)) for v in pair]
          + [v for pair in zip(range(32, 48), range(48, 64)) for v in pair])
_MW2_PERM = (list(range(0, 32, 2)) + list(range(1, 32, 2))
             + list(range(32, 64, 2)) + list(range(33, 64, 2)))

NP = 10240
EP = 327680
NW = 32
EPT = EP // NW
K2 = 256
CH2 = EPT // K2
NPT = NP // 16


def _prep_body(x_ref, w_ref, asw_ref, adw_ref, p_ref, h_ref, hb_ref,
               as_ref, ad_ref, c_ref):
    h = jnp.dot(x_ref[...], w_ref[...], preferred_element_type=F32)
    h_ref[...] = h
    hb_ref[...] = jnp.dot(h, p_ref[...],
                          preferred_element_type=F32).astype(BF16)
    asv = jnp.sum(h * asw_ref[...], axis=1, keepdims=True)
    adv = jnp.sum(h * adw_ref[...], axis=1, keepdims=True)
    as_ref[...] = asv
    ad_ref[...] = adv
    cval = jnp.max(asv) + jnp.max(adv)
    cval = jnp.maximum(cval, 0.2 * cval)
    c_ref[...] = jnp.full((1, 16), cval, F32)


def _prep(x, w, a_src, a_dst):
    return pl.pallas_call(
        _prep_body,
        out_shape=[
            jax.ShapeDtypeStruct((NP, 64), F32),
            jax.ShapeDtypeStruct((NP, 64), BF16),
            jax.ShapeDtypeStruct((NP, 1), F32),
            jax.ShapeDtypeStruct((NP, 1), F32),
            jax.ShapeDtypeStruct((1, 16), F32),
        ],
    )(x, w, a_src.reshape(1, 64), a_dst.reshape(1, 64),
      jnp.eye(64, dtype=F32)[:, jnp.array(_SIGMA)])


def _combine_body(nnodes, matmuls, acc_ref, den_ref, h_ref, as_ref, ad_ref,
                  c_ref, b_ref, *rest):
    w_refs = rest[:matmuls]
    out_refs = rest[matmuls:matmuls + matmuls]
    cval = c_ref[0, 0]
    s = as_ref[...] + ad_ref[...]
    wself = jnp.exp(jnp.maximum(s, 0.2 * s) - cval)
    dent = lax.dot_general(den_ref[...], jnp.ones((NW, 1), F32),
                           (((0,), (0,)), ((), ())),
                           preferred_element_type=F32)
    dent = dent + wself + 1e-16
    a = acc_ref[...]
    acct = a[0] + a[1] + wself * h_ref[...]
    g = jnp.maximum(acct / dent + b_ref[...], 0.0)
    rows = lax.broadcasted_iota(I32, (NP, 1), 0)
    g = jnp.where(rows < nnodes, g, 0.0)
    for w_ref, out_ref in zip(w_refs, out_refs):
        out_ref[...] = jnp.dot(g, w_ref[...], preferred_element_type=F32)


def _combine_next(nnodes, acc, den, h, asv, adv, c, b, w2, a_src2, a_dst2):
    def body(acc_ref, den_ref, h_ref, as_ref, ad_ref, c_ref, b_ref, w_ref,
             asw_ref, adw_ref, p_ref, hn_ref, hnb_ref, as2_ref, ad2_ref,
             c2_ref):
        cval = c_ref[0, 0]
        s = as_ref[...] + ad_ref[...]
        wself = jnp.exp(jnp.maximum(s, 0.2 * s) - cval)
        dent = lax.dot_general(den_ref[...], jnp.ones((NW, 1), F32),
                               (((0,), (0,)), ((), ())),
                               preferred_element_type=F32)
        dent = dent + wself + 1e-16
        a = acc_ref[...]
        acct = a[0] + a[1] + wself * h_ref[...]
        g = jnp.maximum(acct / dent + b_ref[...], 0.0)
        rows = lax.broadcasted_iota(I32, (NP, 1), 0)
        g = jnp.where(rows < nnodes, g, 0.0)
        hn = jnp.dot(g, w_ref[...], preferred_element_type=F32)
        hn_ref[...] = hn
        hnb_ref[...] = jnp.dot(hn, p_ref[...],
                               preferred_element_type=F32).astype(BF16)
        asv2 = jnp.sum(hn * asw_ref[...], axis=1, keepdims=True)
        adv2 = jnp.sum(hn * adw_ref[...], axis=1, keepdims=True)
        as2_ref[...] = asv2
        ad2_ref[...] = adv2
        cv2 = jnp.max(asv2) + jnp.max(adv2)
        cv2 = jnp.maximum(cv2, 0.2 * cv2)
        c2_ref[...] = jnp.full((1, 16), cv2, F32)

    return pl.pallas_call(
        body,
        out_shape=[
            jax.ShapeDtypeStruct((NP, 64), F32),
            jax.ShapeDtypeStruct((NP, 64), BF16),
            jax.ShapeDtypeStruct((NP, 1), F32),
            jax.ShapeDtypeStruct((NP, 1), F32),
            jax.ShapeDtypeStruct((1, 16), F32),
        ],
    )(acc, den, h, asv, adv, c, b.reshape(1, 64), w2,
      a_src2.reshape(1, 64), a_dst2.reshape(1, 64),
      jnp.eye(64, dtype=F32)[:, jnp.array(_SIGMA)])


def _combine_proj(nnodes, acc, den, h, asv, adv, c, b, w_u, w_v):
    def body(acc_ref, den_ref, h_ref, as_ref, ad_ref, c_ref, b_ref,
             wu_ref, wv_ref, g_ref, pu_ref, pv_ref):
        cval = c_ref[0, 0]
        s = as_ref[...] + ad_ref[...]
        wself = jnp.exp(jnp.maximum(s, 0.2 * s) - cval)
        dent = lax.dot_general(den_ref[...], jnp.ones((NW, 1), F32),
                               (((0,), (0,)), ((), ())),
                               preferred_element_type=F32)
        dent = dent + wself + 1e-16
        a = acc_ref[...]
        acct = a[0] + a[1] + wself * h_ref[...]
        g = jnp.maximum(acct / dent + b_ref[...], 0.0)
        rows = lax.broadcasted_iota(I32, (NP, 1), 0)
        g = jnp.where(rows < nnodes, g, 0.0)
        g_ref[...] = g
        pu_ref[...] = jnp.dot(
            g, wu_ref[...], preferred_element_type=F32).astype(BF16)
        pv_ref[...] = jnp.dot(
            g, wv_ref[...], preferred_element_type=F32).astype(BF16)

    return pl.pallas_call(
        body,
        out_shape=[
            jax.ShapeDtypeStruct((NP, 64), F32),
            jax.ShapeDtypeStruct((NP, 64), BF16),
            jax.ShapeDtypeStruct((NP, 64), BF16),
        ],
    )(acc, den, h, asv, adv, c, b.reshape(1, 64), w_u, w_v)


def _pe_body(ea_ref, w_ref, b_ref, o_ref):
    o_ref[...] = (jnp.dot(ea_ref[...], w_ref[...],
                          preferred_element_type=F32)
                  + b_ref[...]).astype(BF16)


def _edge_proj(ea_pad, w_e, mb1):
    bm = 10240
    return pl.pallas_call(
        _pe_body,
        grid=(EP // bm,),
        in_specs=[pl.BlockSpec((bm, 16), lambda i: (i, 0)),
                  pl.BlockSpec((16, 64), lambda i: (0, 0)),
                  pl.BlockSpec((1, 64), lambda i: (0, 0))],
        out_specs=pl.BlockSpec((bm, 64), lambda i: (i, 0)),
        out_shape=jax.ShapeDtypeStruct((EP, 64), BF16),
    )(ea_pad, w_e, mb1.reshape(1, 64))


_MESH = plsc.VectorSubcoreMesh(core_axis_name="c", subcore_axis_name="s")
_SC_PARAMS = pltpu.CompilerParams(needs_layout_passes=False,
                                  use_tc_tiling_on_sc=False)


@functools.partial(
    pl.kernel,
    out_type=[
        jax.ShapeDtypeStruct((2, NP, 64), F32),
        jax.ShapeDtypeStruct((NW, NP), F32),
    ],
    mesh=_MESH,
    scratch_types=[
        pltpu.VMEM((4, 2, 2, 128), I32),
        pltpu.VMEM((4, 2, 2, 128), I32),
        pltpu.VMEM((K2, 64), BF16),
        pltpu.VMEM((K2, 64), BF16),
        pltpu.VMEM((K2, 64), F32),
        pltpu.VMEM((K2, 64), F32),
        pltpu.VMEM((NP,), F32),
        pltpu.VMEM((NP,), F32),
        pltpu.VMEM((NP,), F32),
        pltpu.VMEM((K2,), F32),
        pltpu.VMEM((16,), F32),
        pltpu.VMEM_SHARED((NP, 64), F32),
        pltpu.SemaphoreType.DMA,
        pltpu.SemaphoreType.DMA,
    ],
    compiler_params=_SC_PARAMS,
)
def _gat_edges(sd_hbm, h_hbm, as_hbm,
               ad_hbm, c_hbm, zrow_hbm, zflat_hbm, acc_out, den_out,
               sda, sdb, rba, rbb, rowsa, rowsb, as_v,
               ad_v, den_v, w_v, c_v, acc_sh, sem, sem2):
    cid = lax.axis_index("c")
    sid = lax.axis_index("s")
    wid = sid * 2 + cid
    sdbufs = (sda, sdb)
    bfbufs = (rba, rbb)
    rowbufs = (rowsa, rowsb)
    NSB = CH2 // 4

    pltpu.sync_copy(as_hbm, as_v)
    pltpu.sync_copy(ad_hbm, ad_v)
    pltpu.sync_copy(c_hbm, c_v)
    pltpu.sync_copy(zflat_hbm, den_v)
    pltpu.sync_copy(zrow_hbm, acc_sh.at[pl.ds(sid * NPT, NPT)])
    plsc.subcore_barrier()
    cvec = c_v[...]

    def fire_gather(B, jrow, p):
        for jsub in range(2):
            pltpu.async_copy(h_hbm.at[sdbufs[B].at[jrow, 0, jsub]],
                             bfbufs[p].at[pl.ds(jsub * 128, 128)], sem)

    def drain_gather(B, jrow, p):
        for jsub in range(2):
            pltpu.make_async_copy(
                h_hbm.at[sdbufs[B].at[jrow, 0, jsub]],
                bfbufs[p].at[pl.ds(jsub * 128, 128)], sem).wait()

    def fire_scatter(B, jrow, p):
        for jsub in range(2):
            pltpu.async_copy(rowbufs[p].at[pl.ds(jsub * 128, 128)],
                             acc_sh.at[sdbufs[B].at[jrow, 1, jsub]], sem2,
                             add=True)

    def drain_scatter(B, jrow, p):
        for jsub in range(2):
            pltpu.make_async_copy(
                rowbufs[p].at[pl.ds(jsub * 128, 128)],
                acc_sh.at[sdbufs[B].at[jrow, 1, jsub]], sem2).wait()

    pltpu.sync_copy(sd_hbm.at[wid * NSB], sdbufs[0])
    fire_gather(0, 0, 0)

    def pair(i, carry):
        for sb2 in range(2):
            B = sb2
            sbi = 2 * i + sb2
            gsb = wid * NSB + sbi
            for j in range(4):
              p = j % 2
              drain_gather(B, j, p)
              if j > 0:
                  drain_scatter(B, j - 1, 1 - p)
              else:
                  @pl.when(sbi > 0)
                  def _():
                      drain_scatter(1 - B, 3, 1 - p)
              if j == 2:
                  @pl.when(sbi + 1 < NSB)
                  def _():
                      pltpu.sync_copy(sd_hbm.at[gsb + 1], sdbufs[1 - B])
              if j < 3:
                  fire_gather(B, j + 1, 1 - p)
              else:
                  @pl.when(sbi + 1 < NSB)
                  def _():
                      fire_gather(1 - B, 0, 1 - p)

              def grp(g, inner):
                jj = g // 8
                oo = (g % 8) * 16
                s16 = sdbufs[B][j, 0, jj, pl.ds(oo, 16)]
                d16 = sdbufs[B][j, 1, jj, pl.ds(oo, 16)]
                a1 = plsc.load_gather(as_v, [s16])
                a2 = plsc.load_gather(ad_v, [d16])
                t = a1 + a2
                e = jnp.maximum(t, 0.2 * t)
                w = jnp.exp(e - cvec)
                plsc.addupdate_scatter(den_v, [d16], w)
                w_v[pl.ds(g * 16, 16)] = w
                return inner

              lax.fori_loop(0, K2 // 16, grp, 0)

              @plsc.parallel_loop(0, K2, unroll=4)
              def _(r):
                wb = plsc.load_gather(w_v, [jnp.full((16,), r, I32)])
                for half in range(2):
                    xh = bfbufs[p][r, pl.ds(half * 32, 32)]
                    ev, od = plsc.unpack(xh, format=plsc.PackFormat.INTERLEAVED)
                    rowbufs[p][r, pl.ds(half * 32, 16)] = ev * wb
                    rowbufs[p][r, pl.ds(half * 32 + 16, 16)] = od * wb
              fire_scatter(B, j, p)
        return carry

    lax.fori_loop(0, NSB // 2, pair, 0)
    drain_scatter(1, 3, 1)
    plsc.subcore_barrier()
    pltpu.sync_copy(acc_sh.at[pl.ds(sid * NPT, NPT)],
                    acc_out.at[cid, pl.ds(sid * NPT, NPT)])
    pltpu.sync_copy(den_v, den_out.at[wid])


@functools.partial(
    pl.kernel,
    out_type=jax.ShapeDtypeStruct((EP,), F32),
    mesh=_MESH,
    scratch_types=[
        pltpu.VMEM((2, 128), I32),
        pltpu.VMEM((2, 128), I32),
        pltpu.VMEM((2, 128), I32),
        pltpu.VMEM((2, 128), I32),
        pltpu.VMEM((K2, 64), BF16),
        pltpu.VMEM((K2, 64), BF16),
        pltpu.VMEM((K2, 64), BF16),
        pltpu.VMEM((K2, 64), BF16),
        pltpu.VMEM((K2, 64), BF16),
        pltpu.VMEM((K2, 64), BF16),
        pltpu.VMEM((64,), F32),
        pltpu.VMEM((16,), F32),
        pltpu.VMEM((K2,), F32),
        pltpu.SemaphoreType.DMA,
    ],
    compiler_params=_SC_PARAMS,
)
def _edge_mlp(src2_hbm, dst2_hbm, pu_hbm, pv_hbm, pe_hbm, w2_hbm, b2_hbm,
              out_hbm, srcma, dstma, srcmb, dstmb, pua, pva, pea, pub, pvb,
              peb, w2_v, b2_v, lo_v, sem):
    cid = lax.axis_index("c")
    sid = lax.axis_index("s")
    wid = sid * 2 + cid
    srcbufs = (srcma, srcmb)
    dstbufs = (dstma, dstmb)
    pubufs = (pua, pub)
    pvbufs = (pva, pvb)
    pebufs = (pea, peb)

    pltpu.sync_copy(w2_hbm, w2_v)
    pltpu.sync_copy(b2_hbm, b2_v)
    mvec = [w2_v[pl.ds(q * 16, 16)] for q in range(4)]
    bvec = b2_v[...]

    def fire(ci, p):
        cg = wid * CH2 + ci
        base = wid * EPT + ci * K2
        pltpu.sync_copy(src2_hbm.at[cg], srcbufs[p])
        pltpu.sync_copy(dst2_hbm.at[cg], dstbufs[p])
        for j in range(2):
            pltpu.async_copy(pu_hbm.at[srcbufs[p].at[j]],
                             pubufs[p].at[pl.ds(j * 128, 128)], sem)
            pltpu.async_copy(pv_hbm.at[dstbufs[p].at[j]],
                             pvbufs[p].at[pl.ds(j * 128, 128)], sem)
        pltpu.async_copy(pe_hbm.at[pl.ds(base, K2)], pebufs[p], sem)

    fire(0, 0)

    def pair(i, carry):
        for b in range(2):
            ci = 2 * i + b
            base = wid * EPT + ci * K2
            for j in range(2):
                pltpu.make_async_copy(
                    pu_hbm.at[srcbufs[b].at[j]],
                    pubufs[b].at[pl.ds(j * 128, 128)], sem).wait()
                pltpu.make_async_copy(
                    pv_hbm.at[dstbufs[b].at[j]],
                    pvbufs[b].at[pl.ds(j * 128, 128)], sem).wait()
            pltpu.make_async_copy(pe_hbm.at[pl.ds(base, K2)],
                                  pebufs[b], sem).wait()

            @pl.when(ci + 1 < CH2)
            def _():
                fire(ci + 1, 1 - b)

            def init(g, inner):
                lo_v[pl.ds(g * 16, 16)] = bvec
                return inner

            lax.fori_loop(0, K2 // 16, init, 0)

            @plsc.parallel_loop(0, K2, unroll=4)
            def _(r):
                acc = None
                for half in range(2):
                    ds = pl.ds(half * 32, 32)
                    ua, ub = plsc.unpack(pubufs[b][r, ds],
                                         format=plsc.PackFormat.INTERLEAVED)
                    va, vb = plsc.unpack(pvbufs[b][r, ds],
                                         format=plsc.PackFormat.INTERLEAVED)
                    ea, eb = plsc.unpack(pebufs[b][r, ds],
                                         format=plsc.PackFormat.INTERLEAVED)
                    for pp, vv, ee, q in ((ua, va, ea, 2 * half),
                                          (ub, vb, eb, 2 * half + 1)):
                        hid = jnp.maximum(pp + vv + ee, 0.0)
                        term = hid * mvec[q]
                        acc = term if acc is None else acc + term
                plsc.addupdate_scatter(lo_v, [jnp.full((16,), r, I32)], acc)
            pltpu.sync_copy(lo_v, out_hbm.at[pl.ds(base, K2)])
        return carry

    lax.fori_loop(0, CH2 // 2, pair, 0)


def kernel(x, edge_index, edge_attr, W1, a_src1, a_dst1, b1, W2, a_src2,
           a_dst2, b2, mW1, mb1, mW2, mb2):
    n = x.shape[0]
    e = edge_index.shape[1]

    src = edge_index[0]
    dst = edge_index[1]
    pad = jnp.full((EP - e,), n, I32)
    srcf = jnp.concatenate([src, pad])
    dstf = jnp.concatenate([dst, pad])
    src2 = srcf.reshape(EP // K2, 2, 128)
    dst2 = dstf.reshape(EP // K2, 2, 128)
    sd = jnp.concatenate([srcf.reshape(EP // (K2 * 4), 4, 1, 2, 128),
                          dstf.reshape(EP // (K2 * 4), 4, 1, 2, 128)],
                         axis=2)

    x_pad = jnp.pad(x, ((0, NP - n), (0, 0)))
    ea_pad = jnp.pad(edge_attr, ((0, EP - e), (0, 0)))
    zrow = jnp.zeros((NPT, 64), F32)
    zflat = jnp.zeros((NP,), F32)

    pe = _edge_proj(ea_pad, mW1[128:], mb1)

    h1, h1b, as1, ad1, c1 = _prep(x_pad, W1, a_src1, a_dst1)
    c1v = c1.reshape(16) + (pe[0, 0] * 0).astype(F32)
    acc1, den1 = _gat_edges(sd, h1b, as1.reshape(NP),
                            ad1.reshape(NP), c1v, zrow, zflat)
    h2, h2b, as2, ad2, c2 = _combine_next(n, acc1, den1, h1, as1, ad1, c1,
                                          b1, W2, a_src2, a_dst2)
    c2v = c2.reshape(16)
    acc2, den2 = _gat_edges(sd, h2b, as2.reshape(NP),
                            ad2.reshape(NP), c2v, zrow, zflat)
    g2, pu, pv = _combine_proj(n, acc2, den2, h2, as2, ad2, c2, b2,
                               mW1[:64], mW1[64:128])
    mw2r = mW2.reshape(64)[jnp.array(_MW2_PERM)]
    logits_pad = _edge_mlp(src2, dst2, pu, pv, pe, mw2r,
                           jnp.full((16,), mb2[0], F32))
    return (logits_pad[:e], g2[:n])


# speedup vs baseline: 1.0737x; 1.0002x over previous
import functools

import jax
import jax.numpy as jnp
from jax import lax
from jax.experimental import pallas as pl
from jax.experimental.pallas import tpu as pltpu
from jax.experimental.pallas import tpu_sc as plsc

F32 = jnp.float32
BF16 = jnp.bfloat16
I32 = jnp.int32

_SIGMA = ([v for pair in zip(range(0, 16), range(16, 32)) for v in pair]
          + [v for pair in zip(range(32, 48), range(48, 64)) for v in pair])
_MW2_PERM = (list(range(0, 32, 2)) + list(range(1, 32, 2))
             + list(range(32, 64, 2)) + list(range(33, 64, 2)))

NP = 10240
EP = 327680
NW = 32
EPT = EP // NW
K2 = 256
CH2 = EPT // K2
NPT = NP // 16


def _prep_body(x_ref, w_ref, asw_ref, adw_ref, p_ref, h_ref, hb_ref,
               as_ref, ad_ref, c_ref):
    h = jnp.dot(x_ref[...], w_ref[...], preferred_element_type=F32)
    h_ref[...] = h
    hb_ref[...] = jnp.dot(h, p_ref[...],
                          preferred_element_type=F32).astype(BF16)
    asv = jnp.sum(h * asw_ref[...], axis=1, keepdims=True)
    adv = jnp.sum(h * adw_ref[...], axis=1, keepdims=True)
    as_ref[...] = asv
    ad_ref[...] = adv
    cval = jnp.max(asv) + jnp.max(adv)
    cval = jnp.maximum(cval, 0.2 * cval)
    c_ref[...] = jnp.full((1, 16), cval, F32)


def _prep(x, w, a_src, a_dst):
    return pl.pallas_call(
        _prep_body,
        out_shape=[
            jax.ShapeDtypeStruct((NP, 64), F32),
            jax.ShapeDtypeStruct((NP, 64), BF16),
            jax.ShapeDtypeStruct((NP, 1), F32),
            jax.ShapeDtypeStruct((NP, 1), F32),
            jax.ShapeDtypeStruct((1, 16), F32),
        ],
    )(x, w, a_src.reshape(1, 64), a_dst.reshape(1, 64),
      jnp.eye(64, dtype=F32)[:, jnp.array(_SIGMA)])


def _combine_next(nnodes, acc, den, h, asv, adv, c, b, w2, a_src2, a_dst2):
    def body(acc_ref, den_ref, h_ref, as_ref, ad_ref, c_ref, b_ref, w_ref,
             asw_ref, adw_ref, p_ref, hn_ref, hnb_ref, as2_ref, ad2_ref,
             c2_ref):
        cval = c_ref[0, 0]
        s = as_ref[...] + ad_ref[...]
        wself = jnp.exp(jnp.maximum(s, 0.2 * s) - cval)
        dent = lax.dot_general(den_ref[...], jnp.ones((NW, 1), F32),
                               (((0,), (0,)), ((), ())),
                               preferred_element_type=F32)
        dent = dent + wself + 1e-16
        a = acc_ref[...]
        acct = a[0] + a[1] + wself * h_ref[...]
        g = jnp.maximum(acct / dent + b_ref[...], 0.0)
        rows = lax.broadcasted_iota(I32, (NP, 1), 0)
        g = jnp.where(rows < nnodes, g, 0.0)
        hn = jnp.dot(g, w_ref[...], preferred_element_type=F32)
        hn_ref[...] = hn
        hnb_ref[...] = jnp.dot(hn, p_ref[...],
                               preferred_element_type=F32).astype(BF16)
        asv2 = jnp.sum(hn * asw_ref[...], axis=1, keepdims=True)
        adv2 = jnp.sum(hn * adw_ref[...], axis=1, keepdims=True)
        as2_ref[...] = asv2
        ad2_ref[...] = adv2
        cv2 = jnp.max(asv2) + jnp.max(adv2)
        cv2 = jnp.maximum(cv2, 0.2 * cv2)
        c2_ref[...] = jnp.full((1, 16), cv2, F32)

    return pl.pallas_call(
        body,
        out_shape=[
            jax.ShapeDtypeStruct((NP, 64), F32),
            jax.ShapeDtypeStruct((NP, 64), BF16),
            jax.ShapeDtypeStruct((NP, 1), F32),
            jax.ShapeDtypeStruct((NP, 1), F32),
            jax.ShapeDtypeStruct((1, 16), F32),
        ],
    )(acc, den, h, asv, adv, c, b.reshape(1, 64), w2,
      a_src2.reshape(1, 64), a_dst2.reshape(1, 64),
      jnp.eye(64, dtype=F32)[:, jnp.array(_SIGMA)])


def _combine_proj(nnodes, acc, den, h, asv, adv, c, b, w_u, w_v):
    def body(acc_ref, den_ref, h_ref, as_ref, ad_ref, c_ref, b_ref,
             wu_ref, wv_ref, g_ref, pu_ref, pv_ref):
        cval = c_ref[0, 0]
        s = as_ref[...] + ad_ref[...]
        wself = jnp.exp(jnp.maximum(s, 0.2 * s) - cval)
        dent = lax.dot_general(den_ref[...], jnp.ones((NW, 1), F32),
                               (((0,), (0,)), ((), ())),
                               preferred_element_type=F32)
        dent = dent + wself + 1e-16
        a = acc_ref[...]
        acct = a[0] + a[1] + wself * h_ref[...]
        g = jnp.maximum(acct / dent + b_ref[...], 0.0)
        rows = lax.broadcasted_iota(I32, (NP, 1), 0)
        g = jnp.where(rows < nnodes, g, 0.0)
        g_ref[...] = g
        pu_ref[...] = jnp.dot(
            g, wu_ref[...], preferred_element_type=F32).astype(BF16)
        pv_ref[...] = jnp.dot(
            g, wv_ref[...], preferred_element_type=F32).astype(BF16)

    return pl.pallas_call(
        body,
        out_shape=[
            jax.ShapeDtypeStruct((NP, 64), F32),
            jax.ShapeDtypeStruct((NP, 64), BF16),
            jax.ShapeDtypeStruct((NP, 64), BF16),
        ],
    )(acc, den, h, asv, adv, c, b.reshape(1, 64), w_u, w_v)


def _pe_body(ea_ref, w_ref, b_ref, o_ref):
    o_ref[...] = (jnp.dot(ea_ref[...], w_ref[...],
                          preferred_element_type=F32)
                  + b_ref[...]).astype(BF16)


def _edge_proj(ea_pad, w_e, mb1):
    bm = 10240
    return pl.pallas_call(
        _pe_body,
        grid=(EP // bm,),
        in_specs=[pl.BlockSpec((bm, 16), lambda i: (i, 0)),
                  pl.BlockSpec((16, 64), lambda i: (0, 0)),
                  pl.BlockSpec((1, 64), lambda i: (0, 0))],
        out_specs=pl.BlockSpec((bm, 64), lambda i: (i, 0)),
        out_shape=jax.ShapeDtypeStruct((EP, 64), BF16),
    )(ea_pad, w_e, mb1.reshape(1, 64))


_MESH = plsc.VectorSubcoreMesh(core_axis_name="c", subcore_axis_name="s")
_SC_PARAMS = pltpu.CompilerParams(needs_layout_passes=False,
                                  use_tc_tiling_on_sc=False)


@functools.partial(
    pl.kernel,
    out_type=[
        jax.ShapeDtypeStruct((2, NP, 64), F32),
        jax.ShapeDtypeStruct((NW, NP), F32),
    ],
    mesh=_MESH,
    scratch_types=[
        pltpu.VMEM((4, 2, 2, 128), I32),
        pltpu.VMEM((4, 2, 2, 128), I32),
        pltpu.VMEM((K2, 64), BF16),
        pltpu.VMEM((K2, 64), BF16),
        pltpu.VMEM((K2, 64), F32),
        pltpu.VMEM((K2, 64), F32),
        pltpu.VMEM((NP,), F32),
        pltpu.VMEM((NP,), F32),
        pltpu.VMEM((NP,), F32),
        pltpu.VMEM((K2,), F32),
        pltpu.VMEM((16,), F32),
        pltpu.VMEM_SHARED((NP, 64), F32),
        pltpu.SemaphoreType.DMA,
        pltpu.SemaphoreType.DMA,
    ],
    compiler_params=_SC_PARAMS,
)
def _gat_edges(sd_hbm, h_hbm, as_hbm,
               ad_hbm, c_hbm, zrow_hbm, zflat_hbm, acc_out, den_out,
               sda, sdb, rba, rbb, rowsa, rowsb, as_v,
               ad_v, den_v, w_v, c_v, acc_sh, sem, sem2):
    cid = lax.axis_index("c")
    sid = lax.axis_index("s")
    wid = sid * 2 + cid
    sdbufs = (sda, sdb)
    bfbufs = (rba, rbb)
    rowbufs = (rowsa, rowsb)
    NSB = CH2 // 4

    pltpu.sync_copy(as_hbm, as_v)
    pltpu.sync_copy(ad_hbm, ad_v)
    pltpu.sync_copy(c_hbm, c_v)
    pltpu.sync_copy(zflat_hbm, den_v)
    pltpu.sync_copy(zrow_hbm, acc_sh.at[pl.ds(sid * NPT, NPT)])
    plsc.subcore_barrier()
    cvec = c_v[...]

    def fire_gather(B, jrow, p):
        for jsub in range(2):
            pltpu.async_copy(h_hbm.at[sdbufs[B].at[jrow, 0, jsub]],
                             bfbufs[p].at[pl.ds(jsub * 128, 128)], sem)

    def drain_gather(B, jrow, p):
        for jsub in range(2):
            pltpu.make_async_copy(
                h_hbm.at[sdbufs[B].at[jrow, 0, jsub]],
                bfbufs[p].at[pl.ds(jsub * 128, 128)], sem).wait()

    def fire_scatter(B, jrow, p):
        for jsub in range(2):
            pltpu.async_copy(rowbufs[p].at[pl.ds(jsub * 128, 128)],
                             acc_sh.at[sdbufs[B].at[jrow, 1, jsub]], sem2,
                             add=True)

    def drain_scatter(B, jrow, p):
        for jsub in range(2):
            pltpu.make_async_copy(
                rowbufs[p].at[pl.ds(jsub * 128, 128)],
                acc_sh.at[sdbufs[B].at[jrow, 1, jsub]], sem2).wait()

    pltpu.sync_copy(sd_hbm.at[wid * NSB], sdbufs[0])
    fire_gather(0, 0, 0)

    def pair(i, carry):
        for sb2 in range(2):
            B = sb2
            sbi = 2 * i + sb2
            gsb = wid * NSB + sbi
            for j in range(4):
              p = j % 2
              drain_gather(B, j, p)
              if j > 0:
                  drain_scatter(B, j - 1, 1 - p)
              else:
                  @pl.when(sbi > 0)
                  def _():
                      drain_scatter(1 - B, 3, 1 - p)
              if j == 2:
                  @pl.when(sbi + 1 < NSB)
                  def _():
                      pltpu.sync_copy(sd_hbm.at[gsb + 1], sdbufs[1 - B])
              if j < 3:
                  fire_gather(B, j + 1, 1 - p)
              else:
                  @pl.when(sbi + 1 < NSB)
                  def _():
                      fire_gather(1 - B, 0, 1 - p)

              def grp(g, inner):
                jj = g // 8
                oo = (g % 8) * 16
                s16 = sdbufs[B][j, 0, jj, pl.ds(oo, 16)]
                d16 = sdbufs[B][j, 1, jj, pl.ds(oo, 16)]
                a1 = plsc.load_gather(as_v, [s16])
                a2 = plsc.load_gather(ad_v, [d16])
                t = a1 + a2
                e = jnp.maximum(t, 0.2 * t)
                w = jnp.exp(e - cvec)
                plsc.addupdate_scatter(den_v, [d16], w)
                w_v[pl.ds(g * 16, 16)] = w
                return inner

              lax.fori_loop(0, K2 // 16, grp, 0)

              @plsc.parallel_loop(0, K2, unroll=4)
              def _(r):
                wb = plsc.load_gather(w_v, [jnp.full((16,), r, I32)])
                for half in range(2):
                    xh = bfbufs[p][r, pl.ds(half * 32, 32)]
                    ev, od = plsc.unpack(xh, format=plsc.PackFormat.INTERLEAVED)
                    rowbufs[p][r, pl.ds(half * 32, 16)] = ev * wb
                    rowbufs[p][r, pl.ds(half * 32 + 16, 16)] = od * wb
              fire_scatter(B, j, p)
        return carry

    lax.fori_loop(0, NSB // 2, pair, 0)
    drain_scatter(1, 3, 1)
    plsc.subcore_barrier()
    pltpu.sync_copy(acc_sh.at[pl.ds(sid * NPT, NPT)],
                    acc_out.at[cid, pl.ds(sid * NPT, NPT)])
    pltpu.sync_copy(den_v, den_out.at[wid])


@functools.partial(
    pl.kernel,
    out_type=jax.ShapeDtypeStruct((EP,), F32),
    mesh=_MESH,
    scratch_types=[
        pltpu.VMEM((2, 128), I32),
        pltpu.VMEM((2, 128), I32),
        pltpu.VMEM((2, 128), I32),
        pltpu.VMEM((2, 128), I32),
        pltpu.VMEM((K2, 64), BF16),
        pltpu.VMEM((K2, 64), BF16),
        pltpu.VMEM((K2, 64), BF16),
        pltpu.VMEM((K2, 64), BF16),
        pltpu.VMEM((K2, 64), BF16),
        pltpu.VMEM((K2, 64), BF16),
        pltpu.VMEM((64,), F32),
        pltpu.VMEM((16,), F32),
        pltpu.VMEM((K2,), F32),
        pltpu.SemaphoreType.DMA,
    ],
    compiler_params=_SC_PARAMS,
)
def _edge_mlp(src2_hbm, dst2_hbm, pu_hbm, pv_hbm, pe_hbm, w2_hbm, b2_hbm,
              out_hbm, srcma, dstma, srcmb, dstmb, pua, pva, pea, pub, pvb,
              peb, w2_v, b2_v, lo_v, sem):
    cid = lax.axis_index("c")
    sid = lax.axis_index("s")
    wid = sid * 2 + cid
    srcbufs = (srcma, srcmb)
    dstbufs = (dstma, dstmb)
    pubufs = (pua, pub)
    pvbufs = (pva, pvb)
    pebufs = (pea, peb)

    pltpu.sync_copy(w2_hbm, w2_v)
    pltpu.sync_copy(b2_hbm, b2_v)
    mvec = [w2_v[pl.ds(q * 16, 16)] for q in range(4)]
    bvec = b2_v[...]

    def fire(ci, p):
        cg = wid * CH2 + ci
        base = wid * EPT + ci * K2
        pltpu.sync_copy(src2_hbm.at[cg], srcbufs[p])
        pltpu.sync_copy(dst2_hbm.at[cg], dstbufs[p])
        for j in range(2):
            pltpu.async_copy(pu_hbm.at[srcbufs[p].at[j]],
                             pubufs[p].at[pl.ds(j * 128, 128)], sem)
            pltpu.async_copy(pv_hbm.at[dstbufs[p].at[j]],
                             pvbufs[p].at[pl.ds(j * 128, 128)], sem)
        pltpu.async_copy(pe_hbm.at[pl.ds(base, K2)], pebufs[p], sem)

    fire(0, 0)

    def pair(i, carry):
        for b in range(2):
            ci = 2 * i + b
            base = wid * EPT + ci * K2
            for j in range(2):
                pltpu.make_async_copy(
                    pu_hbm.at[srcbufs[b].at[j]],
                    pubufs[b].at[pl.ds(j * 128, 128)], sem).wait()
                pltpu.make_async_copy(
                    pv_hbm.at[dstbufs[b].at[j]],
                    pvbufs[b].at[pl.ds(j * 128, 128)], sem).wait()
            pltpu.make_async_copy(pe_hbm.at[pl.ds(base, K2)],
                                  pebufs[b], sem).wait()

            @pl.when(ci + 1 < CH2)
            def _():
                fire(ci + 1, 1 - b)

            def init(g, inner):
                lo_v[pl.ds(g * 16, 16)] = bvec
                return inner

            lax.fori_loop(0, K2 // 16, init, 0)

            @plsc.parallel_loop(0, K2, unroll=4)
            def _(r):
                acc = None
                for half in range(2):
                    ds = pl.ds(half * 32, 32)
                    ua, ub = plsc.unpack(pubufs[b][r, ds],
                                         format=plsc.PackFormat.INTERLEAVED)
                    va, vb = plsc.unpack(pvbufs[b][r, ds],
                                         format=plsc.PackFormat.INTERLEAVED)
                    ea, eb = plsc.unpack(pebufs[b][r, ds],
                                         format=plsc.PackFormat.INTERLEAVED)
                    for pp, vv, ee, q in ((ua, va, ea, 2 * half),
                                          (ub, vb, eb, 2 * half + 1)):
                        hid = jnp.maximum(pp + vv + ee, 0.0)
                        term = hid * mvec[q]
                        acc = term if acc is None else acc + term
                plsc.addupdate_scatter(lo_v, [jnp.full((16,), r, I32)], acc)
            pltpu.sync_copy(lo_v, out_hbm.at[pl.ds(base, K2)])
        return carry

    lax.fori_loop(0, CH2 // 2, pair, 0)


def kernel(x, edge_index, edge_attr, W1, a_src1, a_dst1, b1, W2, a_src2,
           a_dst2, b2, mW1, mb1, mW2, mb2):
    n = x.shape[0]
    e = edge_index.shape[1]

    src = edge_index[0]
    dst = edge_index[1]
    pad = jnp.full((EP - e,), n, I32)
    srcf = jnp.concatenate([src, pad])
    dstf = jnp.concatenate([dst, pad])
    src2 = srcf.reshape(EP // K2, 2, 128)
    dst2 = dstf.reshape(EP // K2, 2, 128)
    sd = jnp.concatenate([srcf.reshape(EP // (K2 * 4), 4, 1, 2, 128),
                          dstf.reshape(EP // (K2 * 4), 4, 1, 2, 128)],
                         axis=2)

    x_pad = jnp.pad(x, ((0, NP - n), (0, 0)))
    ea_pad = jnp.pad(edge_attr, ((0, EP - e), (0, 0)))
    zrow = jnp.zeros((NPT, 64), F32)
    zflat = jnp.zeros((NP,), F32)

    pe = _edge_proj(ea_pad, mW1[128:], mb1)

    h1, h1b, as1, ad1, c1 = _prep(x_pad, W1, a_src1, a_dst1)
    c1v = c1.reshape(16) + (pe[0, 0] * 0).astype(F32)
    acc1, den1 = _gat_edges(sd, h1b, as1.reshape(NP),
                            ad1.reshape(NP), c1v, zrow, zflat)
    h2, h2b, as2, ad2, c2 = _combine_next(n, acc1, den1, h1, as1, ad1, c1,
                                          b1, W2, a_src2, a_dst2)
    c2v = c2.reshape(16)
    acc2, den2 = _gat_edges(sd, h2b, as2.reshape(NP),
                            ad2.reshape(NP), c2v, zrow, zflat)
    g2, pu, pv = _combine_proj(n, acc2, den2, h2, as2, ad2, c2, b2,
                               mW1[:64], mW1[64:128])
    mw2r = mW2.reshape(64)[jnp.array(_MW2_PERM)]
    logits_pad = _edge_mlp(src2, dst2, pu, pv, pe, mw2r,
                           jnp.full((16,), mb2[0], F32))
    return (logits_pad[:e], g2[:n])
